# Initial kernel scaffold; baseline (speedup 1.0000x reference)
import jax
import jax.numpy as jnp
from jax.experimental import pallas as pl


def kernel(node_features, edge_features, edge_index, global_features, timestep, params):
    raise NotImplementedError("write your pallas kernel here")


# speedup vs baseline: 1.7101x; 1.7101x over previous
import functools

import jax
import jax.numpy as jnp
import numpy as np
from jax import lax
from jax.experimental import pallas as pl
from jax.experimental.pallas import tpu as pltpu
from jax.experimental.pallas import tpu_sc as plsc

N = 10000
E = 160000
H = 128
HH = 64
EF = 16
MAXN = 50
NPAD = 10240

F32 = jnp.float32

NC = 2
NS = 16
L = 16
NW = NC * NS
EPW = E // NW
CH = 40
NCHUNK = EPW // CH


def _dot_t(a, b):
    return lax.dot_general(a, b, (((1,), (1,)), ((), ())),
                           preferred_element_type=F32)


def _dot(a, b):
    return lax.dot_general(a, b, (((1,), (0,)), ((), ())),
                           preferred_element_type=F32)


def _ln(x, g, b, eps=1e-5):
    m = jnp.mean(x, axis=-1, keepdims=True)
    v = jnp.mean((x - m) ** 2, axis=-1, keepdims=True)
    return (x - m) * lax.rsqrt(v + eps) * g + b


def _silu(x):
    return x * jax.nn.sigmoid(x)


_BN = 2000
_BQ = 200
_BE = 4000


def _k_prologue(nf_ref, w_ref, b_ref, tw1_ref, tb1_ref, tw2_ref, tb2_ref,
                t_ref, out_ref):
    t = t_ref[0, 0]
    t1 = t * tw1_ref[...] + tb1_ref[...]
    temb = _dot_t(_silu(t1), tw2_ref[...]) + tb2_ref[...]
    out_ref[...] = _dot_t(nf_ref[...], w_ref[...]) + b_ref[...] + temb


def _prologue(nf, w, b, tw1, tb1, tw2, tb2, t):
    spec_w = pl.BlockSpec((H, H), lambda i: (0, 0))
    spec_r = pl.BlockSpec((1, H), lambda i: (0, 0))
    return pl.pallas_call(
        _k_prologue,
        grid=(N // _BN,),
        in_specs=[pl.BlockSpec((_BN, H), lambda i: (i, 0)), spec_w, spec_r,
                  spec_r, spec_r, spec_w, spec_r,
                  pl.BlockSpec((1, 1), lambda i: (0, 0))],
        out_specs=pl.BlockSpec((_BN, H), lambda i: (i, 0)),
        out_shape=jax.ShapeDtypeStruct((N, H), F32),
    )(nf, w, b, tw1, tb1, tw2, tb2, t)


def _k_proj5(x_ref, wq, bq, wk, bk, wv, bv, w1, w2, q_o, k_o, v_o, s_o, d_o):
    x = x_ref[...]
    q_o[...] = _dot_t(x, wq[...]) + bq[...]
    k_o[...] = _dot_t(x, wk[...]) + bk[...]
    v_o[...] = _dot_t(x, wv[...]) + bv[...]
    s_o[...] = _dot_t(x, w1[...])
    d_o[...] = _dot_t(x, w2[...])


def _proj5(x, wq, bq, wk, bk, wv, bv, w1, w2):
    spec_w = pl.BlockSpec((H, H), lambda i: (0, 0))
    spec_r = pl.BlockSpec((1, H), lambda i: (0, 0))
    spec_x = pl.BlockSpec((_BN, H), lambda i: (i, 0))
    out = jax.ShapeDtypeStruct((N, H), F32)
    return pl.pallas_call(
        _k_proj5,
        grid=(N // _BN,),
        in_specs=[spec_x, spec_w, spec_r, spec_w, spec_r, spec_w, spec_r,
                  spec_w, spec_w],
        out_specs=[spec_x] * 5,
        out_shape=[out] * 5,
    )(x, wq, bq, wk, bk, wv, bv, w1, w2)


def _k_attn(q_ref, k_ref, v_ref, o_ref):
    s = _dot_t(q_ref[...], k_ref[...]) * (1.0 / np.sqrt(H))
    m = jnp.max(s, axis=-1, keepdims=True)
    e = jnp.exp(s - m)
    a = _dot(e, v_ref[...])
    o_ref[...] = a / jnp.sum(e, axis=-1, keepdims=True)


def _attention(q, k, v):
    return pl.pallas_call(
        _k_attn,
        grid=(N // _BQ,),
        in_specs=[pl.BlockSpec((_BQ, H), lambda i: (i, 0)),
                  pl.BlockSpec((N, H), lambda i: (0, 0)),
                  pl.BlockSpec((N, H), lambda i: (0, 0))],
        out_specs=pl.BlockSpec((_BQ, H), lambda i: (i, 0)),
        out_shape=jax.ShapeDtypeStruct((N, H), F32),
    )(q, k, v)


def _k_msg(pre_ref, ef_ref, c_ref, d_ref, g_ref, b_ref, out_ref):
    eterm = _dot_t(ef_ref[...], c_ref[...]) + d_ref[...]
    out_ref[...] = jnp.maximum(
        _ln(pre_ref[...] + eterm, g_ref[...], b_ref[...]), 0.0)


def _msg_fuse(pre, ef, c, d, g, b):
    spec_r = pl.BlockSpec((1, H), lambda i: (0, 0))
    return pl.pallas_call(
        _k_msg,
        grid=(E // _BE,),
        in_specs=[pl.BlockSpec((_BE, H), lambda i: (i, 0)),
                  pl.BlockSpec((_BE, EF), lambda i: (i, 0)),
                  pl.BlockSpec((H, EF), lambda i: (0, 0)),
                  spec_r, spec_r, spec_r],
        out_specs=pl.BlockSpec((_BE, H), lambda i: (i, 0)),
        out_shape=jax.ShapeDtypeStruct((E, H), F32),
    )(pre, ef, c, d, g, b)


def _k_combine(a_ref, n0_ref, n1_ref, x_ref, wc_ref, u2_ref, c_ref,
               g_ref, b_ref, out_ref):
    acc = _dot_t(a_ref[...], wc_ref[...])
    acc = acc + _dot_t(n0_ref[...] + n1_ref[...], u2_ref[...])
    acc = acc + c_ref[...] + x_ref[...]
    out_ref[...] = _ln(acc, g_ref[...], b_ref[...])


def _combine(a, nm, x, wc, u2, c, g, b):
    spec_w = pl.BlockSpec((H, H), lambda i: (0, 0))
    spec_r = pl.BlockSpec((1, H), lambda i: (0, 0))
    spec_x = pl.BlockSpec((_BN, H), lambda i: (i, 0))
    return pl.pallas_call(
        _k_combine,
        grid=(N // _BN,),
        in_specs=[spec_x, spec_x, spec_x, spec_x, spec_w, spec_w, spec_r,
                  spec_r, spec_r],
        out_specs=spec_x,
        out_shape=jax.ShapeDtypeStruct((N, H), F32),
    )(a, nm[:N], nm[NPAD:NPAD + N], x, wc, u2, c, g, b)


def _k_heads(x_ref, wn1, bn1, ng, nb, wn2, bn2, we1, be1, nn_o, g_o):
    x = x_ref[...]
    h1 = _silu(_ln(_dot_t(x, wn1[...]) + bn1[...], ng[...], nb[...]))
    nn_o[...] = _dot_t(h1, wn2[...]) + bn2[...]
    gv = _dot_t(x, we1[...]) + be1[...]
    g_o[...] = jnp.concatenate([gv, jnp.zeros_like(gv)], axis=1)


def _heads(x, wn1, bn1, ng, nb, wn2, bn2, we1, be1):
    spec_x = pl.BlockSpec((_BN, H), lambda i: (i, 0))
    spec_h = pl.BlockSpec((HH, H), lambda i: (0, 0))
    spec_rh = pl.BlockSpec((1, HH), lambda i: (0, 0))
    return pl.pallas_call(
        _k_heads,
        grid=(N // _BN,),
        in_specs=[spec_x, spec_h, spec_rh, spec_rh, spec_rh,
                  pl.BlockSpec((H, HH), lambda i: (0, 0)),
                  pl.BlockSpec((1, H), lambda i: (0, 0)), spec_h, spec_rh],
        out_specs=[spec_x, spec_x],
        out_shape=[jax.ShapeDtypeStruct((N, H), F32),
                   jax.ShapeDtypeStruct((N, H), F32)],
    )(x, wn1, bn1, ng, nb, wn2, bn2, we1, be1)


def _k_edge_head(p_ref, g_ref, b_ref, w_ref, bo_ref, out_ref):
    h = _silu(_ln(p_ref[:, :HH], g_ref[...], b_ref[...]))
    out_ref[...] = _dot_t(h, w_ref[...]) + bo_ref[...]


def _edge_head(p, g, b, w, bo):
    spec_rh = pl.BlockSpec((1, HH), lambda i: (0, 0))
    return pl.pallas_call(
        _k_edge_head,
        grid=(E // _BE,),
        in_specs=[pl.BlockSpec((_BE, H), lambda i: (i, 0)), spec_rh, spec_rh,
                  pl.BlockSpec((EF, HH), lambda i: (0, 0)),
                  pl.BlockSpec((1, EF), lambda i: (0, 0))],
        out_specs=pl.BlockSpec((_BE, EF), lambda i: (i, 0)),
        out_shape=jax.ShapeDtypeStruct((E, EF), F32),
    )(p, g, b, w, bo)


def _k_topo(x_ref, w1_ref, b1_ref, w2_ref, b2_ref, out_ref):
    gr = jnp.sum(x_ref[...], axis=0, keepdims=True) * (1.0 / N)
    h1 = jnp.maximum(_dot_t(gr, w1_ref[...]) + b1_ref[...], 0.0)
    out_ref[...] = _dot_t(h1, w2_ref[...]) + b2_ref[...]


def _topo(x, w1, b1, w2, b2):
    return pl.pallas_call(
        _k_topo,
        out_shape=jax.ShapeDtypeStruct((1, MAXN * MAXN), F32),
    )(x, w1, b1, w2, b2)


def _sc_pair_gather(table_a, table_b, idx_a, idx_b):
    d = table_a.shape[1]
    mesh = plsc.VectorSubcoreMesh(core_axis_name="c", subcore_axis_name="s")

    @functools.partial(
        pl.kernel, mesh=mesh,
        out_type=jax.ShapeDtypeStruct((E, d), F32),
        scratch_types=[pltpu.VMEM((CH,), jnp.int32),
                       pltpu.VMEM((CH,), jnp.int32),
                       pltpu.VMEM((CH, d), F32),
                       pltpu.VMEM((CH, d), F32),
                       pltpu.SemaphoreType.DMA,
                       pltpu.SemaphoreType.DMA],
    )
    def k(ta, tb, ia, ib, out, iva, ivb, ra, rb, sa, sb):
        wid = lax.axis_index("s") * NC + lax.axis_index("c")
        base = wid * EPW

        def body(j, carry):
            off = base + j * CH
            pltpu.sync_copy(ia.at[pl.ds(off, CH)], iva)
            pltpu.sync_copy(ib.at[pl.ds(off, CH)], ivb)
            ca = pltpu.async_copy(ta.at[iva], ra, sa)
            cb = pltpu.async_copy(tb.at[ivb], rb, sb)
            ca.wait()
            cb.wait()

            def add_row(i, c2):
                for jj in range(d // L):
                    sl = pl.ds(jj * L, L)
                    ra[i, sl] = ra[i, sl] + rb[i, sl]
                return c2

            lax.fori_loop(0, CH, add_row, 0)
            pltpu.sync_copy(ra, out.at[pl.ds(off, CH)])
            return carry

        lax.fori_loop(0, NCHUNK, body, 0)

    return k(table_a, table_b, idx_a, idx_b)


_NPT = NPAD // NS


_HALF = 5120
_TRASH = _HALF
_ACCR = _HALF + 8
_CHP = CH + 8
_ZC = 32
_SPT = _HALF // NS


def _sc_scatter_add(vals, idx):
    mesh = plsc.VectorSubcoreMesh(core_axis_name="c", subcore_axis_name="s")

    @functools.partial(
        pl.kernel, mesh=mesh,
        out_type=jax.ShapeDtypeStruct((NC * NPAD, H), F32),
        scratch_types=[pltpu.VMEM((_CHP,), jnp.int32),
                       pltpu.VMEM((_CHP,), jnp.int32),
                       pltpu.VMEM((_CHP, H), F32),
                       pltpu.VMEM((_ZC,), jnp.int32),
                       pltpu.VMEM((_ZC, H), F32),
                       pltpu.VMEM_SHARED((_ACCR, H), F32),
                       pltpu.SemaphoreType.DMA],
    )
    def k(v, ia, out, iv, ivs, rv, ivz, zbuf, acc, sem):
        cid = lax.axis_index("c")
        sid = lax.axis_index("s")
        wid = sid * NC + cid
        base = wid * EPW
        zeros = jnp.zeros((L,), F32)
        iota = lax.iota(jnp.int32, L)

        def zrow(i, c2):
            for jj in range(H // L):
                zbuf[i, pl.ds(jj * L, L)] = zeros
            return c2

        def phase(p):
            plo = p * _HALF
            lax.fori_loop(0, _ZC, zrow, 0)
            def zchunk(j, c2):
                b = sid * _SPT + j * _ZC
                for off in (0, L):
                    ivz[pl.ds(off, L)] = b + off + iota
                pltpu.sync_copy(zbuf, acc.at[ivz])
                return c2

            lax.fori_loop(0, _SPT // _ZC, zchunk, 0)

            @pl.when(sid == 0)
            def _():
                ivz[pl.ds(0, L)] = _TRASH + jnp.remainder(iota, 8)
                ivz[pl.ds(L, L)] = _TRASH + jnp.remainder(iota, 8)
                pltpu.sync_copy(zbuf, acc.at[ivz])

            plsc.subcore_barrier()

            def body(j, carry):
                off = base + j * CH
                pltpu.sync_copy(ia.at[pl.ds(off, CH)], iv.at[pl.ds(0, CH)])
                pltpu.sync_copy(v.at[pl.ds(off, CH)], rv.at[pl.ds(0, CH)])
                for vv in range(_CHP // L):
                    c = iv[pl.ds(vv * L, L)] - plo
                    ok = (c >= 0) & (c < _HALF)
                    if vv == _CHP // L - 1:
                        ok = ok & (iota < L - 8)
                    ivs[pl.ds(vv * L, L)] = jnp.where(ok, c, _TRASH)
                pltpu.sync_copy(rv, acc.at[ivs], add=True)
                return carry

            lax.fori_loop(0, NCHUNK, body, 0)
            plsc.subcore_barrier()

            def wchunk(j, c2):
                b = sid * _SPT + j * _ZC
                for off in (0, L):
                    ivz[pl.ds(off, L)] = b + off + iota
                pltpu.async_copy(acc.at[ivz], zbuf, sem).wait()
                pltpu.sync_copy(zbuf, out.at[pl.ds(cid * NPAD + plo + b,
                                                   _ZC)])
                return c2

            lax.fori_loop(0, _SPT // _ZC, wchunk, 0)
            plsc.subcore_barrier()

        phase(0)
        phase(1)

    return k(vals, idx)


def kernel(node_features, edge_features, edge_index, global_features,
           timestep, params):
    p = params
    row = edge_index[0]
    col = edge_index[1]
    ef = edge_features
    t = timestep.astype(F32).reshape(1, 1)

    wp, bp = p["edge_proj"]["w"], p["edge_proj"]["b"]

    x = _prologue(
        node_features, p["node_proj"]["w"],
        p["node_proj"]["b"].reshape(1, H),
        p["time_l1"]["w"].reshape(1, H),
        p["time_l1"]["b"].reshape(1, H),
        p["time_l2"]["w"], p["time_l2"]["b"].reshape(1, H), t)

    for lp in p["layers"]:
        mw = lp["mlp_lin"]["w"]
        w1, w2, w3 = mw[:, :H], mw[:, H:2 * H], mw[:, 2 * H:]
        c_mat = w3 @ wp
        d_vec = (w3 @ bp + lp["mlp_lin"]["b"]).reshape(1, H)
        wout = lp["out"]["w"]
        u1, u2 = wout[:, :H], wout[:, H:]
        wc = u1 @ lp["o"]["w"]
        c_vec = (lp["o"]["b"] @ u1.T + lp["out"]["b"]).reshape(1, H)

        q, k, v, xs, xd = _proj5(
            x, lp["q"]["w"], lp["q"]["b"].reshape(1, H),
            lp["k"]["w"], lp["k"]["b"].reshape(1, H),
            lp["v"]["w"], lp["v"]["b"].reshape(1, H), w1, w2)

        a = _attention(q, k, v)
        pre = _sc_pair_gather(xs, xd, row, col)
        msg = _msg_fuse(pre, ef, c_mat, d_vec,
                        lp["mlp_ln"]["g"].reshape(1, H),
                        lp["mlp_ln"]["b"].reshape(1, H))
        nm = _sc_scatter_add(msg, col)
        x = _combine(a, nm, x, wc, u2, c_vec,
                     lp["ln"]["g"].reshape(1, H),
                     lp["ln"]["b"].reshape(1, H))

    node_noise, g = _heads(
        x, p["node_out1"]["w"], p["node_out1"]["b"].reshape(1, HH),
        p["node_out_ln"]["g"].reshape(1, HH),
        p["node_out_ln"]["b"].reshape(1, HH),
        p["node_out2"]["w"], p["node_out2"]["b"].reshape(1, H),
        p["edge_out1"]["w"], p["edge_out1"]["b"].reshape(1, HH))

    pe = _sc_pair_gather(g, g, row, col)
    edge_noise = _edge_head(
        pe, p["edge_out_ln"]["g"].reshape(1, HH),
        p["edge_out_ln"]["b"].reshape(1, HH),
        p["edge_out2"]["w"], p["edge_out2"]["b"].reshape(1, EF))

    topo = _topo(x, p["topo1"]["w"], p["topo1"]["b"].reshape(1, HH),
                 p["topo2"]["w"], p["topo2"]["b"].reshape(1, MAXN * MAXN))

    return (node_noise, edge_noise, topo.reshape(MAXN, MAXN), x)


# speedup vs baseline: 1.7103x; 1.0001x over previous
import functools

import jax
import jax.numpy as jnp
import numpy as np
from jax import lax
from jax.experimental import pallas as pl
from jax.experimental.pallas import tpu as pltpu
from jax.experimental.pallas import tpu_sc as plsc

N = 10000
E = 160000
H = 128
HH = 64
EF = 16
MAXN = 50
NPAD = 10240

F32 = jnp.float32

NC = 2
NS = 16
L = 16
NW = NC * NS
EPW = E // NW
CH = 40
NCHUNK = EPW // CH


def _dot_t(a, b):
    return lax.dot_general(a, b, (((1,), (1,)), ((), ())),
                           preferred_element_type=F32)


def _dot(a, b):
    return lax.dot_general(a, b, (((1,), (0,)), ((), ())),
                           preferred_element_type=F32)


def _ln(x, g, b, eps=1e-5):
    m = jnp.mean(x, axis=-1, keepdims=True)
    v = jnp.mean((x - m) ** 2, axis=-1, keepdims=True)
    return (x - m) * lax.rsqrt(v + eps) * g + b


def _silu(x):
    return x * jax.nn.sigmoid(x)


_BN = 2000
_BQ = 200
_BE = 4000


def _k_prologue(nf_ref, w_ref, b_ref, tw1_ref, tb1_ref, tw2_ref, tb2_ref,
                t_ref, out_ref):
    t = t_ref[0, 0]
    t1 = t * tw1_ref[...] + tb1_ref[...]
    temb = _dot_t(_silu(t1), tw2_ref[...]) + tb2_ref[...]
    out_ref[...] = _dot_t(nf_ref[...], w_ref[...]) + b_ref[...] + temb


def _prologue(nf, w, b, tw1, tb1, tw2, tb2, t):
    spec_w = pl.BlockSpec((H, H), lambda i: (0, 0))
    spec_r = pl.BlockSpec((1, H), lambda i: (0, 0))
    return pl.pallas_call(
        _k_prologue,
        grid=(N // _BN,),
        in_specs=[pl.BlockSpec((_BN, H), lambda i: (i, 0)), spec_w, spec_r,
                  spec_r, spec_r, spec_w, spec_r,
                  pl.BlockSpec((1, 1), lambda i: (0, 0))],
        out_specs=pl.BlockSpec((_BN, H), lambda i: (i, 0)),
        out_shape=jax.ShapeDtypeStruct((N, H), F32),
    )(nf, w, b, tw1, tb1, tw2, tb2, t)


def _k_proj5(x_ref, wq, bq, wk, bk, wv, bv, w1, w2, q_o, k_o, v_o, s_o, d_o):
    x = x_ref[...]
    q_o[...] = _dot_t(x, wq[...]) + bq[...]
    k_o[...] = _dot_t(x, wk[...]) + bk[...]
    v_o[...] = _dot_t(x, wv[...]) + bv[...]
    s_o[...] = _dot_t(x, w1[...])
    d_o[...] = _dot_t(x, w2[...])


def _proj5(x, wq, bq, wk, bk, wv, bv, w1, w2):
    spec_w = pl.BlockSpec((H, H), lambda i: (0, 0))
    spec_r = pl.BlockSpec((1, H), lambda i: (0, 0))
    spec_x = pl.BlockSpec((_BN, H), lambda i: (i, 0))
    out = jax.ShapeDtypeStruct((N, H), F32)
    return pl.pallas_call(
        _k_proj5,
        grid=(N // _BN,),
        in_specs=[spec_x, spec_w, spec_r, spec_w, spec_r, spec_w, spec_r,
                  spec_w, spec_w],
        out_specs=[spec_x] * 5,
        out_shape=[out] * 5,
    )(x, wq, bq, wk, bk, wv, bv, w1, w2)


def _k_attn(q_ref, k_ref, v_ref, o_ref):
    s = _dot_t(q_ref[...].astype(jnp.bfloat16),
               k_ref[...].astype(jnp.bfloat16)) * (1.0 / np.sqrt(H))
    m = jnp.max(s, axis=-1, keepdims=True)
    e = jnp.exp(s - m)
    a = _dot(e.astype(jnp.bfloat16), v_ref[...].astype(jnp.bfloat16))
    o_ref[...] = a / jnp.sum(e, axis=-1, keepdims=True)


def _attention(q, k, v):
    return pl.pallas_call(
        _k_attn,
        grid=(N // _BQ,),
        in_specs=[pl.BlockSpec((_BQ, H), lambda i: (i, 0)),
                  pl.BlockSpec((N, H), lambda i: (0, 0)),
                  pl.BlockSpec((N, H), lambda i: (0, 0))],
        out_specs=pl.BlockSpec((_BQ, H), lambda i: (i, 0)),
        out_shape=jax.ShapeDtypeStruct((N, H), F32),
    )(q, k, v)


def _k_msg(pre_ref, ef_ref, c_ref, d_ref, g_ref, b_ref, out_ref):
    eterm = _dot_t(ef_ref[...], c_ref[...]) + d_ref[...]
    out_ref[...] = jnp.maximum(
        _ln(pre_ref[...] + eterm, g_ref[...], b_ref[...]), 0.0)


def _msg_fuse(pre, ef, c, d, g, b):
    spec_r = pl.BlockSpec((1, H), lambda i: (0, 0))
    return pl.pallas_call(
        _k_msg,
        grid=(E // _BE,),
        in_specs=[pl.BlockSpec((_BE, H), lambda i: (i, 0)),
                  pl.BlockSpec((_BE, EF), lambda i: (i, 0)),
                  pl.BlockSpec((H, EF), lambda i: (0, 0)),
                  spec_r, spec_r, spec_r],
        out_specs=pl.BlockSpec((_BE, H), lambda i: (i, 0)),
        out_shape=jax.ShapeDtypeStruct((E, H), F32),
    )(pre, ef, c, d, g, b)


def _k_combine(a_ref, n0_ref, n1_ref, x_ref, wc_ref, u2_ref, c_ref,
               g_ref, b_ref, out_ref):
    acc = _dot_t(a_ref[...], wc_ref[...])
    acc = acc + _dot_t(n0_ref[...] + n1_ref[...], u2_ref[...])
    acc = acc + c_ref[...] + x_ref[...]
    out_ref[...] = _ln(acc, g_ref[...], b_ref[...])


def _combine(a, nm, x, wc, u2, c, g, b):
    spec_w = pl.BlockSpec((H, H), lambda i: (0, 0))
    spec_r = pl.BlockSpec((1, H), lambda i: (0, 0))
    spec_x = pl.BlockSpec((_BN, H), lambda i: (i, 0))
    return pl.pallas_call(
        _k_combine,
        grid=(N // _BN,),
        in_specs=[spec_x, spec_x, spec_x, spec_x, spec_w, spec_w, spec_r,
                  spec_r, spec_r],
        out_specs=spec_x,
        out_shape=jax.ShapeDtypeStruct((N, H), F32),
    )(a, nm[:N], nm[NPAD:NPAD + N], x, wc, u2, c, g, b)


def _k_heads(x_ref, wn1, bn1, ng, nb, wn2, bn2, we1, be1, nn_o, g_o):
    x = x_ref[...]
    h1 = _silu(_ln(_dot_t(x, wn1[...]) + bn1[...], ng[...], nb[...]))
    nn_o[...] = _dot_t(h1, wn2[...]) + bn2[...]
    gv = _dot_t(x, we1[...]) + be1[...]
    g_o[...] = jnp.concatenate([gv, jnp.zeros_like(gv)], axis=1)


def _heads(x, wn1, bn1, ng, nb, wn2, bn2, we1, be1):
    spec_x = pl.BlockSpec((_BN, H), lambda i: (i, 0))
    spec_h = pl.BlockSpec((HH, H), lambda i: (0, 0))
    spec_rh = pl.BlockSpec((1, HH), lambda i: (0, 0))
    return pl.pallas_call(
        _k_heads,
        grid=(N // _BN,),
        in_specs=[spec_x, spec_h, spec_rh, spec_rh, spec_rh,
                  pl.BlockSpec((H, HH), lambda i: (0, 0)),
                  pl.BlockSpec((1, H), lambda i: (0, 0)), spec_h, spec_rh],
        out_specs=[spec_x, spec_x],
        out_shape=[jax.ShapeDtypeStruct((N, H), F32),
                   jax.ShapeDtypeStruct((N, H), F32)],
    )(x, wn1, bn1, ng, nb, wn2, bn2, we1, be1)


def _k_edge_head(p_ref, g_ref, b_ref, w_ref, bo_ref, out_ref):
    h = _silu(_ln(p_ref[:, :HH], g_ref[...], b_ref[...]))
    out_ref[...] = _dot_t(h, w_ref[...]) + bo_ref[...]


def _edge_head(p, g, b, w, bo):
    spec_rh = pl.BlockSpec((1, HH), lambda i: (0, 0))
    return pl.pallas_call(
        _k_edge_head,
        grid=(E // _BE,),
        in_specs=[pl.BlockSpec((_BE, H), lambda i: (i, 0)), spec_rh, spec_rh,
                  pl.BlockSpec((EF, HH), lambda i: (0, 0)),
                  pl.BlockSpec((1, EF), lambda i: (0, 0))],
        out_specs=pl.BlockSpec((_BE, EF), lambda i: (i, 0)),
        out_shape=jax.ShapeDtypeStruct((E, EF), F32),
    )(p, g, b, w, bo)


def _k_topo(x_ref, w1_ref, b1_ref, w2_ref, b2_ref, out_ref):
    gr = jnp.sum(x_ref[...], axis=0, keepdims=True) * (1.0 / N)
    h1 = jnp.maximum(_dot_t(gr, w1_ref[...]) + b1_ref[...], 0.0)
    out_ref[...] = _dot_t(h1, w2_ref[...]) + b2_ref[...]


def _topo(x, w1, b1, w2, b2):
    return pl.pallas_call(
        _k_topo,
        out_shape=jax.ShapeDtypeStruct((1, MAXN * MAXN), F32),
    )(x, w1, b1, w2, b2)


def _sc_pair_gather(table_a, table_b, idx_a, idx_b):
    d = table_a.shape[1]
    mesh = plsc.VectorSubcoreMesh(core_axis_name="c", subcore_axis_name="s")

    @functools.partial(
        pl.kernel, mesh=mesh,
        out_type=jax.ShapeDtypeStruct((E, d), F32),
        scratch_types=[pltpu.VMEM((CH,), jnp.int32),
                       pltpu.VMEM((CH,), jnp.int32),
                       pltpu.VMEM((CH, d), F32),
                       pltpu.VMEM((CH, d), F32),
                       pltpu.SemaphoreType.DMA,
                       pltpu.SemaphoreType.DMA],
    )
    def k(ta, tb, ia, ib, out, iva, ivb, ra, rb, sa, sb):
        wid = lax.axis_index("s") * NC + lax.axis_index("c")
        base = wid * EPW

        def body(j, carry):
            off = base + j * CH
            pltpu.sync_copy(ia.at[pl.ds(off, CH)], iva)
            pltpu.sync_copy(ib.at[pl.ds(off, CH)], ivb)
            ca = pltpu.async_copy(ta.at[iva], ra, sa)
            cb = pltpu.async_copy(tb.at[ivb], rb, sb)
            ca.wait()
            cb.wait()

            def add_row(i, c2):
                for jj in range(d // L):
                    sl = pl.ds(jj * L, L)
                    ra[i, sl] = ra[i, sl] + rb[i, sl]
                return c2

            lax.fori_loop(0, CH, add_row, 0)
            pltpu.sync_copy(ra, out.at[pl.ds(off, CH)])
            return carry

        lax.fori_loop(0, NCHUNK, body, 0)

    return k(table_a, table_b, idx_a, idx_b)


_NPT = NPAD // NS


_HALF = 5120
_TRASH = _HALF
_ACCR = _HALF + 8
_CHP = CH + 8
_ZC = 32
_SPT = _HALF // NS


def _sc_scatter_add(vals, idx):
    mesh = plsc.VectorSubcoreMesh(core_axis_name="c", subcore_axis_name="s")

    @functools.partial(
        pl.kernel, mesh=mesh,
        out_type=jax.ShapeDtypeStruct((NC * NPAD, H), F32),
        scratch_types=[pltpu.VMEM((_CHP,), jnp.int32),
                       pltpu.VMEM((_CHP,), jnp.int32),
                       pltpu.VMEM((_CHP, H), F32),
                       pltpu.VMEM((_ZC,), jnp.int32),
                       pltpu.VMEM((_ZC, H), F32),
                       pltpu.VMEM_SHARED((_ACCR, H), F32),
                       pltpu.SemaphoreType.DMA],
    )
    def k(v, ia, out, iv, ivs, rv, ivz, zbuf, acc, sem):
        cid = lax.axis_index("c")
        sid = lax.axis_index("s")
        wid = sid * NC + cid
        base = wid * EPW
        zeros = jnp.zeros((L,), F32)
        iota = lax.iota(jnp.int32, L)

        def zrow(i, c2):
            for jj in range(H // L):
                zbuf[i, pl.ds(jj * L, L)] = zeros
            return c2

        def phase(p):
            plo = p * _HALF
            lax.fori_loop(0, _ZC, zrow, 0)
            def zchunk(j, c2):
                b = sid * _SPT + j * _ZC
                for off in (0, L):
                    ivz[pl.ds(off, L)] = b + off + iota
                pltpu.sync_copy(zbuf, acc.at[ivz])
                return c2

            lax.fori_loop(0, _SPT // _ZC, zchunk, 0)

            @pl.when(sid == 0)
            def _():
                ivz[pl.ds(0, L)] = _TRASH + jnp.remainder(iota, 8)
                ivz[pl.ds(L, L)] = _TRASH + jnp.remainder(iota, 8)
                pltpu.sync_copy(zbuf, acc.at[ivz])

            plsc.subcore_barrier()

            def body(j, carry):
                off = base + j * CH
                pltpu.sync_copy(ia.at[pl.ds(off, CH)], iv.at[pl.ds(0, CH)])
                pltpu.sync_copy(v.at[pl.ds(off, CH)], rv.at[pl.ds(0, CH)])
                for vv in range(_CHP // L):
                    c = iv[pl.ds(vv * L, L)] - plo
                    ok = (c >= 0) & (c < _HALF)
                    if vv == _CHP // L - 1:
                        ok = ok & (iota < L - 8)
                    ivs[pl.ds(vv * L, L)] = jnp.where(ok, c, _TRASH)
                pltpu.sync_copy(rv, acc.at[ivs], add=True)
                return carry

            lax.fori_loop(0, NCHUNK, body, 0)
            plsc.subcore_barrier()

            def wchunk(j, c2):
                b = sid * _SPT + j * _ZC
                for off in (0, L):
                    ivz[pl.ds(off, L)] = b + off + iota
                pltpu.async_copy(acc.at[ivz], zbuf, sem).wait()
                pltpu.sync_copy(zbuf, out.at[pl.ds(cid * NPAD + plo + b,
                                                   _ZC)])
                return c2

            lax.fori_loop(0, _SPT // _ZC, wchunk, 0)
            plsc.subcore_barrier()

        phase(0)
        phase(1)

    return k(vals, idx)


def kernel(node_features, edge_features, edge_index, global_features,
           timestep, params):
    p = params
    row = edge_index[0]
    col = edge_index[1]
    ef = edge_features
    t = timestep.astype(F32).reshape(1, 1)

    wp, bp = p["edge_proj"]["w"], p["edge_proj"]["b"]

    x = _prologue(
        node_features, p["node_proj"]["w"],
        p["node_proj"]["b"].reshape(1, H),
        p["time_l1"]["w"].reshape(1, H),
        p["time_l1"]["b"].reshape(1, H),
        p["time_l2"]["w"], p["time_l2"]["b"].reshape(1, H), t)

    for lp in p["layers"]:
        mw = lp["mlp_lin"]["w"]
        w1, w2, w3 = mw[:, :H], mw[:, H:2 * H], mw[:, 2 * H:]
        c_mat = w3 @ wp
        d_vec = (w3 @ bp + lp["mlp_lin"]["b"]).reshape(1, H)
        wout = lp["out"]["w"]
        u1, u2 = wout[:, :H], wout[:, H:]
        wc = u1 @ lp["o"]["w"]
        c_vec = (lp["o"]["b"] @ u1.T + lp["out"]["b"]).reshape(1, H)

        q, k, v, xs, xd = _proj5(
            x, lp["q"]["w"], lp["q"]["b"].reshape(1, H),
            lp["k"]["w"], lp["k"]["b"].reshape(1, H),
            lp["v"]["w"], lp["v"]["b"].reshape(1, H), w1, w2)

        a = _attention(q, k, v)
        pre = _sc_pair_gather(xs, xd, row, col)
        msg = _msg_fuse(pre, ef, c_mat, d_vec,
                        lp["mlp_ln"]["g"].reshape(1, H),
                        lp["mlp_ln"]["b"].reshape(1, H))
        nm = _sc_scatter_add(msg, col)
        x = _combine(a, nm, x, wc, u2, c_vec,
                     lp["ln"]["g"].reshape(1, H),
                     lp["ln"]["b"].reshape(1, H))

    node_noise, g = _heads(
        x, p["node_out1"]["w"], p["node_out1"]["b"].reshape(1, HH),
        p["node_out_ln"]["g"].reshape(1, HH),
        p["node_out_ln"]["b"].reshape(1, HH),
        p["node_out2"]["w"], p["node_out2"]["b"].reshape(1, H),
        p["edge_out1"]["w"], p["edge_out1"]["b"].reshape(1, HH))

    pe = _sc_pair_gather(g, g, row, col)
    edge_noise = _edge_head(
        pe, p["edge_out_ln"]["g"].reshape(1, HH),
        p["edge_out_ln"]["b"].reshape(1, HH),
        p["edge_out2"]["w"], p["edge_out2"]["b"].reshape(1, EF))

    topo = _topo(x, p["topo1"]["w"], p["topo1"]["b"].reshape(1, HH),
                 p["topo2"]["w"], p["topo2"]["b"].reshape(1, MAXN * MAXN))

    return (node_noise, edge_noise, topo.reshape(MAXN, MAXN), x)


# speedup vs baseline: 2.1797x; 1.2744x over previous
import functools

import jax
import jax.numpy as jnp
import numpy as np
from jax import lax
from jax.experimental import pallas as pl
from jax.experimental.pallas import tpu as pltpu
from jax.experimental.pallas import tpu_sc as plsc

N = 10000
E = 160000
H = 128
HH = 64
EF = 16
MAXN = 50
NPAD = 10240

F32 = jnp.float32

NC = 2
NS = 16
L = 16
NW = NC * NS
EPW = E // NW
CH = 40
NCHUNK = EPW // CH


def _dot_t(a, b):
    return lax.dot_general(a, b, (((1,), (1,)), ((), ())),
                           preferred_element_type=F32)


def _dot(a, b):
    return lax.dot_general(a, b, (((1,), (0,)), ((), ())),
                           preferred_element_type=F32)


def _ln(x, g, b, eps=1e-5):
    m = jnp.mean(x, axis=-1, keepdims=True)
    v = jnp.mean((x - m) ** 2, axis=-1, keepdims=True)
    return (x - m) * lax.rsqrt(v + eps) * g + b


def _silu(x):
    return x * jax.nn.sigmoid(x)


_BN = 2000
_BQ = 200
_BE = 4000


def _k_prologue(nf_ref, w_ref, b_ref, tw1_ref, tb1_ref, tw2_ref, tb2_ref,
                t_ref, out_ref):
    t = t_ref[0, 0]
    t1 = t * tw1_ref[...] + tb1_ref[...]
    temb = _dot_t(_silu(t1), tw2_ref[...]) + tb2_ref[...]
    out_ref[...] = _dot_t(nf_ref[...], w_ref[...]) + b_ref[...] + temb


def _prologue(nf, w, b, tw1, tb1, tw2, tb2, t):
    spec_w = pl.BlockSpec((H, H), lambda i: (0, 0))
    spec_r = pl.BlockSpec((1, H), lambda i: (0, 0))
    return pl.pallas_call(
        _k_prologue,
        grid=(N // _BN,),
        in_specs=[pl.BlockSpec((_BN, H), lambda i: (i, 0)), spec_w, spec_r,
                  spec_r, spec_r, spec_w, spec_r,
                  pl.BlockSpec((1, 1), lambda i: (0, 0))],
        out_specs=pl.BlockSpec((_BN, H), lambda i: (i, 0)),
        out_shape=jax.ShapeDtypeStruct((N, H), F32),
    )(nf, w, b, tw1, tb1, tw2, tb2, t)


def _k_proj5(x_ref, wq, bq, wk, bk, wv, bv, w1, w2, q_o, k_o, v_o, s_o, d_o):
    x = x_ref[...]
    q_o[...] = _dot_t(x, wq[...]) + bq[...]
    k_o[...] = _dot_t(x, wk[...]) + bk[...]
    v_o[...] = _dot_t(x, wv[...]) + bv[...]
    s_o[...] = _dot_t(x, w1[...])
    d_o[...] = _dot_t(x, w2[...])


def _proj5(x, wq, bq, wk, bk, wv, bv, w1, w2):
    spec_w = pl.BlockSpec((H, H), lambda i: (0, 0))
    spec_r = pl.BlockSpec((1, H), lambda i: (0, 0))
    spec_x = pl.BlockSpec((_BN, H), lambda i: (i, 0))
    out = jax.ShapeDtypeStruct((N, H), F32)
    return pl.pallas_call(
        _k_proj5,
        grid=(N // _BN,),
        in_specs=[spec_x, spec_w, spec_r, spec_w, spec_r, spec_w, spec_r,
                  spec_w, spec_w],
        out_specs=[spec_x] * 5,
        out_shape=[out] * 5,
    )(x, wq, bq, wk, bk, wv, bv, w1, w2)


def _k_attn(q_ref, k_ref, v_ref, o_ref):
    s = _dot_t(q_ref[...].astype(jnp.bfloat16),
               k_ref[...].astype(jnp.bfloat16)) * (1.0 / np.sqrt(H))
    m = jnp.max(s, axis=-1, keepdims=True)
    e = jnp.exp(s - m)
    a = _dot(e.astype(jnp.bfloat16), v_ref[...].astype(jnp.bfloat16))
    o_ref[...] = a / jnp.sum(e, axis=-1, keepdims=True)


def _attention(q, k, v):
    return pl.pallas_call(
        _k_attn,
        grid=(N // _BQ,),
        in_specs=[pl.BlockSpec((_BQ, H), lambda i: (i, 0)),
                  pl.BlockSpec((N, H), lambda i: (0, 0)),
                  pl.BlockSpec((N, H), lambda i: (0, 0))],
        out_specs=pl.BlockSpec((_BQ, H), lambda i: (i, 0)),
        out_shape=jax.ShapeDtypeStruct((N, H), F32),
    )(q, k, v)


def _k_msg(pre_ref, ef_ref, c_ref, d_ref, g_ref, b_ref, out_ref):
    eterm = _dot_t(ef_ref[...], c_ref[...]) + d_ref[...]
    out_ref[...] = jnp.maximum(
        _ln(pre_ref[...] + eterm, g_ref[...], b_ref[...]), 0.0)


def _msg_fuse(pre, ef, c, d, g, b):
    spec_r = pl.BlockSpec((1, H), lambda i: (0, 0))
    return pl.pallas_call(
        _k_msg,
        grid=(E // _BE,),
        in_specs=[pl.BlockSpec((_BE, H), lambda i: (i, 0)),
                  pl.BlockSpec((_BE, EF), lambda i: (i, 0)),
                  pl.BlockSpec((H, EF), lambda i: (0, 0)),
                  spec_r, spec_r, spec_r],
        out_specs=pl.BlockSpec((_BE, H), lambda i: (i, 0)),
        out_shape=jax.ShapeDtypeStruct((E, H), F32),
    )(pre, ef, c, d, g, b)


def _k_combine(a_ref, n0_ref, n1_ref, x_ref, wc_ref, u2_ref, c_ref,
               g_ref, b_ref, out_ref):
    acc = _dot_t(a_ref[...], wc_ref[...])
    acc = acc + _dot_t(n0_ref[...] + n1_ref[...], u2_ref[...])
    acc = acc + c_ref[...] + x_ref[...]
    out_ref[...] = _ln(acc, g_ref[...], b_ref[...])


def _combine(a, nm, x, wc, u2, c, g, b):
    spec_w = pl.BlockSpec((H, H), lambda i: (0, 0))
    spec_r = pl.BlockSpec((1, H), lambda i: (0, 0))
    spec_x = pl.BlockSpec((_BN, H), lambda i: (i, 0))
    return pl.pallas_call(
        _k_combine,
        grid=(N // _BN,),
        in_specs=[spec_x, spec_x, spec_x, spec_x, spec_w, spec_w, spec_r,
                  spec_r, spec_r],
        out_specs=spec_x,
        out_shape=jax.ShapeDtypeStruct((N, H), F32),
    )(a, nm[:N], nm[NPAD:NPAD + N], x, wc, u2, c, g, b)


def _k_heads(x_ref, wn1, bn1, ng, nb, wn2, bn2, we1, be1, nn_o, g_o):
    x = x_ref[...]
    h1 = _silu(_ln(_dot_t(x, wn1[...]) + bn1[...], ng[...], nb[...]))
    nn_o[...] = _dot_t(h1, wn2[...]) + bn2[...]
    gv = _dot_t(x, we1[...]) + be1[...]
    g_o[...] = jnp.concatenate([gv, jnp.zeros_like(gv)], axis=1)


def _heads(x, wn1, bn1, ng, nb, wn2, bn2, we1, be1):
    spec_x = pl.BlockSpec((_BN, H), lambda i: (i, 0))
    spec_h = pl.BlockSpec((HH, H), lambda i: (0, 0))
    spec_rh = pl.BlockSpec((1, HH), lambda i: (0, 0))
    return pl.pallas_call(
        _k_heads,
        grid=(N // _BN,),
        in_specs=[spec_x, spec_h, spec_rh, spec_rh, spec_rh,
                  pl.BlockSpec((H, HH), lambda i: (0, 0)),
                  pl.BlockSpec((1, H), lambda i: (0, 0)), spec_h, spec_rh],
        out_specs=[spec_x, spec_x],
        out_shape=[jax.ShapeDtypeStruct((N, H), F32),
                   jax.ShapeDtypeStruct((N, H), F32)],
    )(x, wn1, bn1, ng, nb, wn2, bn2, we1, be1)


def _k_edge_head(p_ref, g_ref, b_ref, w_ref, bo_ref, out_ref):
    h = _silu(_ln(p_ref[:, :HH], g_ref[...], b_ref[...]))
    out_ref[...] = _dot_t(h, w_ref[...]) + bo_ref[...]


def _edge_head(p, g, b, w, bo):
    spec_rh = pl.BlockSpec((1, HH), lambda i: (0, 0))
    return pl.pallas_call(
        _k_edge_head,
        grid=(E // _BE,),
        in_specs=[pl.BlockSpec((_BE, H), lambda i: (i, 0)), spec_rh, spec_rh,
                  pl.BlockSpec((EF, HH), lambda i: (0, 0)),
                  pl.BlockSpec((1, EF), lambda i: (0, 0))],
        out_specs=pl.BlockSpec((_BE, EF), lambda i: (i, 0)),
        out_shape=jax.ShapeDtypeStruct((E, EF), F32),
    )(p, g, b, w, bo)


def _k_topo(x_ref, w1_ref, b1_ref, w2_ref, b2_ref, out_ref):
    gr = jnp.sum(x_ref[...], axis=0, keepdims=True) * (1.0 / N)
    h1 = jnp.maximum(_dot_t(gr, w1_ref[...]) + b1_ref[...], 0.0)
    out_ref[...] = _dot_t(h1, w2_ref[...]) + b2_ref[...]


def _topo(x, w1, b1, w2, b2):
    return pl.pallas_call(
        _k_topo,
        out_shape=jax.ShapeDtypeStruct((1, MAXN * MAXN), F32),
    )(x, w1, b1, w2, b2)


def _sc_pair_gather(table_a, table_b, idx_a, idx_b):
    d = table_a.shape[1]
    mesh = plsc.VectorSubcoreMesh(core_axis_name="c", subcore_axis_name="s")

    @functools.partial(
        pl.kernel, mesh=mesh,
        out_type=jax.ShapeDtypeStruct((E, d), F32),
        scratch_types=[[pltpu.VMEM((CH,), jnp.int32)] * 2,
                       [pltpu.VMEM((CH,), jnp.int32)] * 2,
                       [pltpu.VMEM((CH, d), F32)] * 2,
                       [pltpu.VMEM((CH, d), F32)] * 2,
                       [pltpu.SemaphoreType.DMA] * 2,
                       [pltpu.SemaphoreType.DMA] * 2],
    )
    def k(ta, tb, ia, ib, out, iva, ivb, ra, rb, si, sg):
        wid = lax.axis_index("s") * NC + lax.axis_index("c")
        base = wid * EPW

        def fire_idx(c, b):
            off = base + c * CH
            pltpu.async_copy(ia.at[pl.ds(off, CH)], iva[b], si[b])
            pltpu.async_copy(ib.at[pl.ds(off, CH)], ivb[b], si[b])

        def wait_idx(c, b):
            off = base + c * CH
            pltpu.make_async_copy(ia.at[pl.ds(off, CH)], iva[b], si[b]).wait()
            pltpu.make_async_copy(ib.at[pl.ds(off, CH)], ivb[b], si[b]).wait()

        def fire_gather(b):
            pltpu.async_copy(ta.at[iva[b]], ra[b], sg[b])
            pltpu.async_copy(tb.at[ivb[b]], rb[b], sg[b])

        def add_write(c, b):
            pltpu.make_async_copy(ta.at[iva[b]], ra[b], sg[b]).wait()
            pltpu.make_async_copy(tb.at[ivb[b]], rb[b], sg[b]).wait()

            def add_row(i, c2):
                for jj in range(d // L):
                    sl = pl.ds(jj * L, L)
                    ra[b][i, sl] = ra[b][i, sl] + rb[b][i, sl]
                return c2

            lax.fori_loop(0, CH, add_row, 0)
            pltpu.sync_copy(ra[b], out.at[pl.ds(base + c * CH, CH)])

        fire_idx(0, 0)

        def step(j, carry):
            c0 = 2 * j
            wait_idx(c0, 0)
            fire_gather(0)

            @pl.when(j > 0)
            def _():
                add_write(c0 - 1, 1)

            fire_idx(c0 + 1, 1)
            wait_idx(c0 + 1, 1)
            fire_gather(1)
            add_write(c0, 0)
            fire_idx(c0 + 2, 0)
            return carry

        lax.fori_loop(0, (NCHUNK - 1) // 2, step, 0)
        wait_idx(NCHUNK - 1, 0)
        fire_gather(0)
        add_write(NCHUNK - 2, 1)
        add_write(NCHUNK - 1, 0)

    return k(table_a, table_b, idx_a, idx_b)


_NPT = NPAD // NS


_HALF = 5120
_TRASH = _HALF
_ACCR = _HALF + 8
_CHP = CH + 8
_ZC = 32
_SPT = _HALF // NS


def _sc_scatter_add(vals, idx):
    mesh = plsc.VectorSubcoreMesh(core_axis_name="c", subcore_axis_name="s")

    @functools.partial(
        pl.kernel, mesh=mesh,
        out_type=jax.ShapeDtypeStruct((NC * NPAD, H), F32),
        scratch_types=[pltpu.VMEM((_CHP,), jnp.int32),
                       pltpu.VMEM((_CHP,), jnp.int32),
                       pltpu.VMEM((_CHP, H), F32),
                       pltpu.VMEM((_ZC,), jnp.int32),
                       pltpu.VMEM((_ZC, H), F32),
                       pltpu.VMEM_SHARED((_ACCR, H), F32),
                       pltpu.SemaphoreType.DMA],
    )
    def k(v, ia, out, iv, ivs, rv, ivz, zbuf, acc, sem):
        cid = lax.axis_index("c")
        sid = lax.axis_index("s")
        wid = sid * NC + cid
        base = wid * EPW
        zeros = jnp.zeros((L,), F32)
        iota = lax.iota(jnp.int32, L)

        def zrow(i, c2):
            for jj in range(H // L):
                zbuf[i, pl.ds(jj * L, L)] = zeros
            return c2

        def phase(p):
            plo = p * _HALF
            lax.fori_loop(0, _ZC, zrow, 0)
            def zchunk(j, c2):
                b = sid * _SPT + j * _ZC
                for off in (0, L):
                    ivz[pl.ds(off, L)] = b + off + iota
                pltpu.sync_copy(zbuf, acc.at[ivz])
                return c2

            lax.fori_loop(0, _SPT // _ZC, zchunk, 0)

            @pl.when(sid == 0)
            def _():
                ivz[pl.ds(0, L)] = _TRASH + jnp.remainder(iota, 8)
                ivz[pl.ds(L, L)] = _TRASH + jnp.remainder(iota, 8)
                pltpu.sync_copy(zbuf, acc.at[ivz])

            plsc.subcore_barrier()

            def body(j, carry):
                off = base + j * CH
                pltpu.sync_copy(ia.at[pl.ds(off, CH)], iv.at[pl.ds(0, CH)])
                pltpu.sync_copy(v.at[pl.ds(off, CH)], rv.at[pl.ds(0, CH)])
                for vv in range(_CHP // L):
                    c = iv[pl.ds(vv * L, L)] - plo
                    ok = (c >= 0) & (c < _HALF)
                    if vv == _CHP // L - 1:
                        ok = ok & (iota < L - 8)
                    ivs[pl.ds(vv * L, L)] = jnp.where(ok, c, _TRASH)
                pltpu.sync_copy(rv, acc.at[ivs], add=True)
                return carry

            lax.fori_loop(0, NCHUNK, body, 0)
            plsc.subcore_barrier()

            def wchunk(j, c2):
                b = sid * _SPT + j * _ZC
                for off in (0, L):
                    ivz[pl.ds(off, L)] = b + off + iota
                pltpu.async_copy(acc.at[ivz], zbuf, sem).wait()
                pltpu.sync_copy(zbuf, out.at[pl.ds(cid * NPAD + plo + b,
                                                   _ZC)])
                return c2

            lax.fori_loop(0, _SPT // _ZC, wchunk, 0)
            plsc.subcore_barrier()

        phase(0)
        phase(1)

    return k(vals, idx)


def kernel(node_features, edge_features, edge_index, global_features,
           timestep, params):
    p = params
    row = edge_index[0]
    col = edge_index[1]
    ef = edge_features
    t = timestep.astype(F32).reshape(1, 1)

    wp, bp = p["edge_proj"]["w"], p["edge_proj"]["b"]

    x = _prologue(
        node_features, p["node_proj"]["w"],
        p["node_proj"]["b"].reshape(1, H),
        p["time_l1"]["w"].reshape(1, H),
        p["time_l1"]["b"].reshape(1, H),
        p["time_l2"]["w"], p["time_l2"]["b"].reshape(1, H), t)

    for lp in p["layers"]:
        mw = lp["mlp_lin"]["w"]
        w1, w2, w3 = mw[:, :H], mw[:, H:2 * H], mw[:, 2 * H:]
        c_mat = w3 @ wp
        d_vec = (w3 @ bp + lp["mlp_lin"]["b"]).reshape(1, H)
        wout = lp["out"]["w"]
        u1, u2 = wout[:, :H], wout[:, H:]
        wc = u1 @ lp["o"]["w"]
        c_vec = (lp["o"]["b"] @ u1.T + lp["out"]["b"]).reshape(1, H)

        q, k, v, xs, xd = _proj5(
            x, lp["q"]["w"], lp["q"]["b"].reshape(1, H),
            lp["k"]["w"], lp["k"]["b"].reshape(1, H),
            lp["v"]["w"], lp["v"]["b"].reshape(1, H), w1, w2)

        a = _attention(q, k, v)
        pre = _sc_pair_gather(xs, xd, row, col)
        msg = _msg_fuse(pre, ef, c_mat, d_vec,
                        lp["mlp_ln"]["g"].reshape(1, H),
                        lp["mlp_ln"]["b"].reshape(1, H))
        nm = _sc_scatter_add(msg, col)
        x = _combine(a, nm, x, wc, u2, c_vec,
                     lp["ln"]["g"].reshape(1, H),
                     lp["ln"]["b"].reshape(1, H))

    node_noise, g = _heads(
        x, p["node_out1"]["w"], p["node_out1"]["b"].reshape(1, HH),
        p["node_out_ln"]["g"].reshape(1, HH),
        p["node_out_ln"]["b"].reshape(1, HH),
        p["node_out2"]["w"], p["node_out2"]["b"].reshape(1, H),
        p["edge_out1"]["w"], p["edge_out1"]["b"].reshape(1, HH))

    pe = _sc_pair_gather(g, g, row, col)
    edge_noise = _edge_head(
        pe, p["edge_out_ln"]["g"].reshape(1, HH),
        p["edge_out_ln"]["b"].reshape(1, HH),
        p["edge_out2"]["w"], p["edge_out2"]["b"].reshape(1, EF))

    topo = _topo(x, p["topo1"]["w"], p["topo1"]["b"].reshape(1, HH),
                 p["topo2"]["w"], p["topo2"]["b"].reshape(1, MAXN * MAXN))

    return (node_noise, edge_noise, topo.reshape(MAXN, MAXN), x)
